# baseline (speedup 1.0000x reference)
import functools

import jax
import jax.numpy as jnp
from jax.experimental import pallas as pl
from jax.experimental.pallas import tpu as pltpu
from jax.experimental.pallas import tpu_sc as plsc

_NC, _NS, _L = 2, 16, 16
_NW = _NC * _NS


def _sc_count_partials(maskf):
    M, S = maskf.shape
    rpw = M // _NW
    nch = S // _L
    mesh = plsc.VectorSubcoreMesh(core_axis_name="c", subcore_axis_name="s")

    @functools.partial(
        pl.kernel, mesh=mesh,
        out_type=jax.ShapeDtypeStruct((M, _L), jnp.float32),
        scratch_types=[
            pltpu.VMEM((rpw, S), jnp.float32),
            pltpu.VMEM((rpw, _L), jnp.float32),
        ],
    )
    def k(mask_hbm, out_hbm, rows_v, part_v):
        wid = jax.lax.axis_index("s") * _NC + jax.lax.axis_index("c")
        base = wid * rpw
        pltpu.sync_copy(mask_hbm.at[pl.ds(base, rpw)], rows_v)
        for r in range(rpw):
            acc = rows_v[r, 0:_L]
            for c in range(1, nch):
                acc = acc + rows_v[r, c * _L:(c + 1) * _L]
            part_v[r, :] = acc
        pltpu.sync_copy(part_v, out_hbm.at[pl.ds(base, rpw)])

    return k(maskf)


def _body(emb_ref, vis_ref, bbox_ref, kp_ref, mask_ref,
          w1evb_ref, w1k_ref, w2a_ref, sel_ref, out_ref, *, T, S, inv_s):
    bf = jnp.bfloat16
    f32 = jnp.float32
    dn = (((1,), (0,)), ((), ()))
    m = mask_ref[...]
    TS = T * S
    EMB = jnp.concatenate([emb_ref[t] for t in range(T)], axis=1)
    BBX = jnp.concatenate([bbox_ref[t] for t in range(T)], axis=1)
    VIS = jnp.concatenate([vis_ref[t] for t in range(T)], axis=1)
    EVB = jnp.concatenate(
        [EMB, BBX, VIS, jnp.ones((1, TS), dtype=f32)], axis=0).astype(bf)
    h = jax.lax.dot_general(
        w1evb_ref[...], EVB, dn, preferred_element_type=f32)
    h += jax.lax.dot_general(
        w1k_ref[...], kp_ref[0].astype(bf), dn, preferred_element_type=f32)
    h = jnp.maximum(h, 0.0).astype(bf)
    mrow = jnp.concatenate(
        [m[t:t + 1, :] for t in range(T)], axis=1).astype(bf)
    hs = jax.lax.dot_general(
        h * mrow, sel_ref[...], dn, preferred_element_type=f32)
    out = jax.lax.dot_general(
        w2a_ref[...], hs.astype(bf), dn,
        preferred_element_type=f32)
    out_ref[...] = out[None] * inv_s


def kernel(embeddings, visibility_scores, bbox_ltwh, keypoints_xyc,
           feats_masks, W1, b1, W2, b2):
    B, N, S, E = embeddings.shape
    KP = keypoints_xyc.shape[3]
    K3 = KP * 3
    M = B * N
    F = W1.shape[1]
    O = W2.shape[0]

    T = 128
    NB = N // T

    embT = embeddings.transpose(0, 1, 3, 2).reshape(M, E, S)
    visT = visibility_scores.transpose(0, 1, 3, 2).reshape(M, 1, S)
    bboxT = bbox_ltwh.transpose(0, 1, 3, 2).reshape(M, 4, S)
    kpT = keypoints_xyc.transpose(0, 3, 4, 1, 2).reshape(B, K3, N * S)
    maskf = feats_masks.astype(jnp.float32).reshape(M, S)

    bf = jnp.bfloat16
    W1evb = jnp.concatenate(
        [W1[:, :E], W1[:, E + 1:E + 5], W1[:, E:E + 1], b1[:, None]],
        axis=1).astype(bf)
    W1k = W1[:, E + 5:].astype(bf)
    W2bf = W2.astype(bf)
    sel01 = (jnp.arange(T * S)[:, None] // S
             == jnp.arange(T)[None, :]).astype(bf)

    grid = (M // T,)
    body = functools.partial(_body, T=T, S=S, inv_s=1.0 / S)
    out = pl.pallas_call(
        body,
        grid=grid,
        in_specs=[
            pl.BlockSpec((T, E, S), lambda i: (i, 0, 0)),
            pl.BlockSpec((T, 1, S), lambda i: (i, 0, 0)),
            pl.BlockSpec((T, 4, S), lambda i: (i, 0, 0)),
            pl.BlockSpec((1, K3, T * S), lambda i: (i // NB, 0, i % NB)),
            pl.BlockSpec((T, S), lambda i: (i, 0)),
            pl.BlockSpec((F, E + 6), lambda i: (0, 0)),
            pl.BlockSpec((F, K3), lambda i: (0, 0)),
            pl.BlockSpec((O, F), lambda i: (0, 0)),
            pl.BlockSpec((T * S, T), lambda i: (0, 0)),
        ],
        out_specs=pl.BlockSpec((1, O, T), lambda i: (i, 0, 0)),
        out_shape=jax.ShapeDtypeStruct((M // T, O, T), jnp.float32),
    )(embT, visT, bboxT, kpT, maskf, W1evb, W1k, W2bf, sel01)
    cnt = _sc_count_partials(maskf).sum(axis=1)
    tokens = out.transpose(0, 2, 1).reshape(B, N, O)
    return tokens + (cnt.reshape(B, N, 1) * b2[None, None, :]) * (1.0 / S)
